# Initial kernel scaffold; baseline (speedup 1.0000x reference)
import jax
import jax.numpy as jnp
from jax.experimental import pallas as pl


def kernel(input_tensor, position_embeddings):
    raise NotImplementedError("write your pallas kernel here")


# speedup vs baseline: 1.6800x; 1.6800x over previous
import jax
import jax.numpy as jnp
from jax.experimental import pallas as pl


_BS = 512


def _body(x_ref, p_ref, o_ref):
    o_ref[...] = x_ref[...] + p_ref[...]


def kernel(input_tensor, position_embeddings):
    B, S, H = input_tensor.shape
    grid = (S // _BS, B)
    return pl.pallas_call(
        _body,
        grid=grid,
        in_specs=[
            pl.BlockSpec((1, _BS, H), lambda s, b: (b, s, 0)),
            pl.BlockSpec((_BS, H), lambda s, b: (s, 0)),
        ],
        out_specs=pl.BlockSpec((1, _BS, H), lambda s, b: (b, s, 0)),
        out_shape=jax.ShapeDtypeStruct((B, S, H), input_tensor.dtype),
    )(input_tensor, position_embeddings)


# speedup vs baseline: 1.8777x; 1.1177x over previous
import jax
import jax.numpy as jnp
from jax.experimental import pallas as pl


_BS = 1024


def _body(x_ref, p_ref, o_ref):
    o_ref[...] = x_ref[...] + p_ref[...]


def kernel(input_tensor, position_embeddings):
    B, S, H = input_tensor.shape
    grid = (S // _BS, B)
    return pl.pallas_call(
        _body,
        grid=grid,
        in_specs=[
            pl.BlockSpec((1, _BS, H), lambda s, b: (b, s, 0)),
            pl.BlockSpec((_BS, H), lambda s, b: (s, 0)),
        ],
        out_specs=pl.BlockSpec((1, _BS, H), lambda s, b: (b, s, 0)),
        out_shape=jax.ShapeDtypeStruct((B, S, H), input_tensor.dtype),
    )(input_tensor, position_embeddings)


# speedup vs baseline: 1.9924x; 1.0611x over previous
import jax
import jax.numpy as jnp
from jax.experimental import pallas as pl


_BS = 2048


def _body(x_ref, p_ref, o_ref):
    o_ref[...] = x_ref[...] + p_ref[...]


def kernel(input_tensor, position_embeddings):
    B, S, H = input_tensor.shape
    grid = (S // _BS, B)
    return pl.pallas_call(
        _body,
        grid=grid,
        in_specs=[
            pl.BlockSpec((1, _BS, H), lambda s, b: (b, s, 0)),
            pl.BlockSpec((_BS, H), lambda s, b: (s, 0)),
        ],
        out_specs=pl.BlockSpec((1, _BS, H), lambda s, b: (b, s, 0)),
        out_shape=jax.ShapeDtypeStruct((B, S, H), input_tensor.dtype),
    )(input_tensor, position_embeddings)
